# baseline (speedup 1.0000x reference)
import functools

import jax
import jax.numpy as jnp
from jax import lax
from jax.experimental import pallas as pl
from jax.experimental.pallas import tpu as pltpu
from jax.experimental.pallas import tpu_sc as plsc

D = 128
H = 4
NC, NS, L = 2, 16, 16
BLK = 2048
F32 = jnp.float32
I32 = jnp.int32
PADV = 1 << 30

_MESH = plsc.VectorSubcoreMesh(core_axis_name="c", subcore_axis_name="s",
                               num_cores=NC, num_subcores=NS)
_SC_PARAMS = pltpu.CompilerParams(needs_layout_passes=False)


def _rup(x, m):
    return (x + m - 1) // m * m


@functools.cache
def _sc_seg_sum(N_src, E_pad, R, n_rng2, want_cnt):
    n_pass = n_rng2 // NC
    CH = E_pad // NS
    n_blk = CH // BLK
    K = 128
    CAP = 2176
    RD = R + 16
    RT = R // NS
    outs = [jax.ShapeDtypeStruct((n_rng2 * R, D), F32)]
    if want_cnt:
        outs.append(jax.ShapeDtypeStruct((NS, n_rng2, RD), F32))
    scratch = [
        pltpu.VMEM((BLK,), I32),
        pltpu.VMEM((BLK,), I32),
        pltpu.VMEM((CAP,), I32),
        pltpu.VMEM((CAP,), I32),
        pltpu.VMEM((1, K), I32),
        pltpu.VMEM((K, D), F32),
        pltpu.VMEM((8, D), F32),
        pltpu.VMEM((RD,), F32),
        pltpu.VMEM_SHARED((R + 1, D), F32),
        pltpu.SemaphoreType.DMA,
    ]

    def body(x_hbm, src_hbm, dst_hbm, *rest):
        if want_cnt:
            out_hbm, cnt_hbm = rest[0], rest[1]
            sbuf, dbuf, gbuf, lbuf, didx, rows, zbuf, cntp, accum, sem = rest[2:]
        else:
            out_hbm = rest[0]
            cnt_hbm = None
            sbuf, dbuf, gbuf, lbuf, didx, rows, zbuf, cntp, accum, sem = rest[1:]
        c = lax.axis_index("c")
        s = lax.axis_index("s")
        iot = lax.iota(I32, L)
        zv = jnp.zeros((L,), F32)
        ones = jnp.ones((L,), F32)

        def _zb(i, _):
            for k in range(D // L):
                zbuf[i, pl.ds(k * L, L)] = zv
            return 0
        lax.fori_loop(0, 8, _zb, 0)

        def do_pass(p, _):
            r = p * NC + c
            lo = r * R

            def _za(i, _):
                pltpu.sync_copy(zbuf, accum.at[pl.ds(s * RT + i * 8, 8)])
                return 0
            lax.fori_loop(0, RT // 8, _za, 0)
            if want_cnt:
                def _zc(i, _):
                    cntp[pl.ds(i * L, L)] = zv
                    return 0
                lax.fori_loop(0, RD // L, _zc, 0)
            plsc.subcore_barrier()

            def do_blk(bi, _):
                base = s * CH + bi * BLK
                pltpu.sync_copy(src_hbm.at[pl.ds(base, BLK)], sbuf)
                pltpu.sync_copy(dst_hbm.at[pl.ds(base, BLK)], dbuf)

                def filt(v, off):
                    dv = dbuf[pl.ds(v * L, L)]
                    sv = sbuf[pl.ds(v * L, L)]
                    m = (dv >= lo) & (dv < lo + R)
                    dl = jnp.where(m, dv - lo, R)
                    if want_cnt:
                        plsc.addupdate_scatter(cntp, [dl], ones, mask=m)
                    plsc.store_compressed(gbuf.at[pl.ds(off, L)], sv, mask=m)
                    plsc.store_compressed(lbuf.at[pl.ds(off, L)], dl, mask=m)
                    return off + jnp.sum(m.astype(I32))
                n = lax.fori_loop(0, BLK // L, filt, jnp.int32(0))
                nb = (n + K - 1) // K

                def padf(v, _):
                    sel = (iot + v * L) >= n
                    gv = gbuf[pl.ds(v * L, L)]
                    lv = lbuf[pl.ds(v * L, L)]
                    gbuf[pl.ds(v * L, L)] = jnp.where(sel, 0, gv)
                    lbuf[pl.ds(v * L, L)] = jnp.where(sel, R, lv)
                    return 0
                lax.fori_loop(n // L, nb * (K // L), padf, 0)

                def batch(b, _):
                    pltpu.async_copy(
                        x_hbm.at[gbuf.at[pl.ds(b * K, K)]], rows, sem).wait()
                    for k in range(K // L):
                        didx[0, pl.ds(k * L, L)] = lbuf[pl.ds(b * K + k * L, L)]
                    pltpu.sync_copy(rows, accum.at[didx.at[0]], add=True)
                    return 0
                lax.fori_loop(0, nb, batch, 0)
                return 0
            lax.fori_loop(0, n_blk, do_blk, 0)
            plsc.subcore_barrier()
            pltpu.sync_copy(accum.at[pl.ds(s * RT, RT)],
                            out_hbm.at[pl.ds(r * R + s * RT, RT)])
            if want_cnt:
                pltpu.sync_copy(cntp, cnt_hbm.at[s, r])
            return 0
        lax.fori_loop(0, n_pass, do_pass, 0)

    return pl.kernel(body, out_type=tuple(outs), mesh=_MESH,
                     scratch_types=scratch, compiler_params=_SC_PARAMS)


@functools.cache
def _sc_hist(N_dst, E_pad, R, n_rng2):
    n_pass = n_rng2 // NC
    CH = E_pad // NS
    n_blk = CH // BLK
    RD = R + 16
    scratch = [
        pltpu.VMEM((BLK,), I32),
        pltpu.VMEM((RD,), F32),
    ]

    def body(dst_hbm, out_hbm, dbuf, cntp):
        c = lax.axis_index("c")
        s = lax.axis_index("s")
        zv = jnp.zeros((L,), F32)
        ones = jnp.ones((L,), F32)

        def do_pass(p, _):
            r = p * NC + c
            lo = r * R

            def _zc(i, _):
                cntp[pl.ds(i * L, L)] = zv
                return 0
            lax.fori_loop(0, RD // L, _zc, 0)

            def do_blk(bi, _):
                base = s * CH + bi * BLK
                pltpu.sync_copy(dst_hbm.at[pl.ds(base, BLK)], dbuf)

                def inner(v, _):
                    dv = dbuf[pl.ds(v * L, L)]
                    m = (dv >= lo) & (dv < lo + R)
                    dl = jnp.where(m, dv - lo, R)
                    plsc.addupdate_scatter(cntp, [dl], ones, mask=m)
                    return 0
                lax.fori_loop(0, BLK // L, inner, 0)
                return 0
            lax.fori_loop(0, n_blk, do_blk, 0)
            pltpu.sync_copy(cntp, out_hbm.at[s, r])
            return 0
        lax.fori_loop(0, n_pass, do_pass, 0)

    return pl.kernel(
        body, out_type=jax.ShapeDtypeStruct((NS, n_rng2, RD), F32),
        mesh=_MESH, scratch_types=scratch, compiler_params=_SC_PARAMS)


@functools.cache
def _sc_gat(N, E_pad, R, n_rng2):
    BLKG = 1024
    n_pass = n_rng2 // NC
    CH = E_pad // NS
    n_blk = CH // BLKG
    K = 64
    CAP = BLKG + 128
    RD = R + 16
    RT = R // NS
    RP = R + 1
    outs = [jax.ShapeDtypeStruct((H, n_rng2 * R, D), F32),
            jax.ShapeDtypeStruct((NS, n_rng2, H * RD), F32)]
    scratch = [
        pltpu.VMEM((BLKG,), I32),
        pltpu.VMEM((BLKG,), I32),
        pltpu.VMEM((CAP,), I32),
        pltpu.VMEM((CAP,), I32),
        pltpu.VMEM((K,), I32),
        pltpu.VMEM((1, K), I32),
        pltpu.VMEM((K, D), F32),
        pltpu.VMEM((K, D), F32),
        pltpu.VMEM((K, D), F32),
        pltpu.VMEM((K, D), F32),
        pltpu.VMEM((8, D), F32),
        pltpu.VMEM((H * RD,), F32),
        pltpu.VMEM((2, D), F32),
        pltpu.VMEM_SHARED((H * RP, D), F32),
        pltpu.SemaphoreType.DMA,
    ]

    def body(x_hbm, as_hbm, ad_hbm, m_hbm, src_hbm, dst_hbm, out_hbm, cnt_hbm,
             sbuf, dbuf, gbuf, lbuf, aidx, didx, rows, wbuf, asb, adb, zbuf,
             cntp, mb, accum, sem):
        c = lax.axis_index("c")
        s = lax.axis_index("s")
        iot = lax.iota(I32, L)
        zv = jnp.zeros((L,), F32)

        def _zb(i, _):
            for k in range(D // L):
                zbuf[i, pl.ds(k * L, L)] = zv
            return 0
        lax.fori_loop(0, 8, _zb, 0)
        pltpu.sync_copy(m_hbm.at[pl.ds(0, 2)], mb)

        def do_pass(p, _):
            r = p * NC + c
            lo = r * R
            for h in range(H):
                def _za(i, _, h=h):
                    pltpu.sync_copy(
                        zbuf, accum.at[pl.ds(h * RP + s * RT + i * 8, 8)])
                    return 0
                lax.fori_loop(0, RT // 8, _za, 0)

            def _zc(i, _):
                cntp[pl.ds(i * L, L)] = zv
                return 0
            lax.fori_loop(0, (H * RD) // L, _zc, 0)
            plsc.subcore_barrier()

            def do_blk(bi, _):
                base = s * CH + bi * BLKG
                pltpu.sync_copy(src_hbm.at[pl.ds(base, BLKG)], sbuf)
                pltpu.sync_copy(dst_hbm.at[pl.ds(base, BLKG)], dbuf)

                def filt(v, off):
                    dv = dbuf[pl.ds(v * L, L)]
                    sv = sbuf[pl.ds(v * L, L)]
                    m = (dv >= lo) & (dv < lo + R)
                    dl = jnp.where(m, dv - lo, R)
                    plsc.store_compressed(gbuf.at[pl.ds(off, L)], sv, mask=m)
                    plsc.store_compressed(lbuf.at[pl.ds(off, L)], dl, mask=m)
                    return off + jnp.sum(m.astype(I32))
                n = lax.fori_loop(0, BLKG // L, filt, jnp.int32(0))
                nb = (n + K - 1) // K

                def padf(v, _):
                    sel = (iot + v * L) >= n
                    gv = gbuf[pl.ds(v * L, L)]
                    lv = lbuf[pl.ds(v * L, L)]
                    gbuf[pl.ds(v * L, L)] = jnp.where(sel, 0, gv)
                    lbuf[pl.ds(v * L, L)] = jnp.where(sel, R, lv)
                    return 0
                lax.fori_loop(n // L, nb * (K // L), padf, 0)

                def batch(b, _):
                    for k in range(K // L):
                        aidx[pl.ds(k * L, L)] = jnp.minimum(
                            lbuf[pl.ds(b * K + k * L, L)] + lo, N - 1)
                    pltpu.async_copy(
                        x_hbm.at[gbuf.at[pl.ds(b * K, K)]], rows, sem).wait()
                    pltpu.async_copy(
                        as_hbm.at[gbuf.at[pl.ds(b * K, K)]], asb, sem).wait()
                    pltpu.async_copy(ad_hbm.at[aidx], adb, sem).wait()
                    for g in range(K // L):
                        rowi = iot + g * L
                        dlv = lbuf[pl.ds(b * K + g * L, L)]
                        for h in range(H):
                            hc = jnp.full((L,), h * L, I32)
                            av = plsc.load_gather(asb, [rowi, hc])
                            bv = plsc.load_gather(adb, [rowi, hc])
                            vv = av + bv
                            e = jnp.maximum(vv, 0.2 * vv)
                            w = jnp.exp(e - (mb[0, pl.ds(h * L, L)]
                                             + mb[1, pl.ds(h * L, L)]))
                            plsc.addupdate_scatter(cntp, [dlv + h * RD], w)
                    for h in range(H):
                        def per_edge(j, _, h=h):
                            va = (asb[j, pl.ds(h * L, L)]
                                  + adb[j, pl.ds(h * L, L)])
                            e = jnp.maximum(va, 0.2 * va)
                            w = jnp.exp(e - (mb[0, pl.ds(h * L, L)]
                                             + mb[1, pl.ds(h * L, L)]))
                            for k in range(D // L):
                                wbuf[j, pl.ds(k * L, L)] = (
                                    rows[j, pl.ds(k * L, L)] * w)
                            return 0
                        lax.fori_loop(0, K, per_edge, 0)
                        for k in range(K // L):
                            didx[0, pl.ds(k * L, L)] = (
                                lbuf[pl.ds(b * K + k * L, L)] + h * RP)
                        pltpu.sync_copy(wbuf, accum.at[didx.at[0]], add=True)
                    return 0
                lax.fori_loop(0, nb, batch, 0)
                return 0
            lax.fori_loop(0, n_blk, do_blk, 0)
            plsc.subcore_barrier()
            for h in range(H):
                pltpu.sync_copy(accum.at[pl.ds(h * RP + s * RT, RT)],
                                out_hbm.at[h, pl.ds(r * R + s * RT, RT)])
            pltpu.sync_copy(cntp, cnt_hbm.at[s, r])
            return 0
        lax.fori_loop(0, n_pass, do_pass, 0)

    return pl.kernel(body, out_type=tuple(outs), mesh=_MESH,
                     scratch_types=scratch, compiler_params=_SC_PARAMS)


def _tc_red(T, X, BX=8192):
    def body(p_ref, o_ref):
        o_ref[...] = jnp.sum(p_ref[...], axis=0)
    return pl.pallas_call(
        body, grid=(pl.cdiv(X, BX),),
        in_specs=[pl.BlockSpec((T, BX), lambda i: (0, i))],
        out_specs=pl.BlockSpec((BX,), lambda i: (i,)),
        out_shape=jax.ShapeDtypeStruct((X,), F32))


def _tc_dinv(X, BX=8192):
    def body(h_ref, o_ref):
        o_ref[...] = lax.rsqrt(h_ref[...] + 1.0)
    return pl.pallas_call(
        body, grid=(pl.cdiv(X, BX),),
        in_specs=[pl.BlockSpec((BX,), lambda i: (i,))],
        out_specs=pl.BlockSpec((BX,), lambda i: (i,)),
        out_shape=jax.ShapeDtypeStruct((X,), F32))


def _tc_scale(N, B=1024):
    def body(d_ref, x_ref, o_ref):
        o_ref[...] = d_ref[...][:, None] * x_ref[...]
    return pl.pallas_call(
        body, grid=(pl.cdiv(N, B),),
        in_specs=[pl.BlockSpec((B,), lambda i: (i,)),
                  pl.BlockSpec((B, D), lambda i: (i, 0))],
        out_specs=pl.BlockSpec((B, D), lambda i: (i, 0)),
        out_shape=jax.ShapeDtypeStruct((N, D), F32))


def _tc_pre(N, B=512):
    g = pl.cdiv(N, B)

    def body(x_ref, ws_ref, wd_ref, ats_ref, atd_ref, as_ref, ad_ref, m_ref):
        i = pl.program_id(0)
        x = x_ref[...]
        rvalid = (lax.broadcasted_iota(I32, (B, 1), 0) + i * B) < N

        def tab(wref, atref):
            cols = []
            for h in range(H):
                wh = wref[:, h * D:(h + 1) * D]
                cols.append(jnp.sum(wh * atref[h * D:(h + 1) * D][None, :],
                                    axis=1, keepdims=True))
            A = jnp.concatenate(cols, axis=1)
            a = jnp.dot(x, A, preferred_element_type=F32)
            parts = [jnp.broadcast_to(a[:, h:h + 1], (B, L)) for h in range(H)]
            parts.append(jnp.zeros((B, D - H * L), F32))
            am = jnp.where(rvalid, a, -1e30)
            mx = jnp.max(am, axis=0)
            mparts = [jnp.broadcast_to(mx[h:h + 1], (L,)) for h in range(H)]
            mparts.append(jnp.zeros((D - H * L,), F32))
            return jnp.concatenate(parts, axis=1), jnp.concatenate(mparts)

        as_tab, ms = tab(ws_ref, ats_ref)
        ad_tab, md = tab(wd_ref, atd_ref)
        as_ref[...] = as_tab
        ad_ref[...] = ad_tab
        mnew = jnp.concatenate(
            [ms[None, :], md[None, :], jnp.full((6, D), -1e30, F32)], axis=0)
        @pl.when(i == 0)
        def _():
            m_ref[...] = jnp.full((8, D), -1e30, F32)
        m_ref[...] = jnp.maximum(m_ref[...], mnew)

    return pl.pallas_call(
        body, grid=(g,),
        in_specs=[pl.BlockSpec((B, D), lambda i: (i, 0)),
                  pl.BlockSpec((D, H * D), lambda i: (0, 0)),
                  pl.BlockSpec((D, H * D), lambda i: (0, 0)),
                  pl.BlockSpec((H * D,), lambda i: (0,)),
                  pl.BlockSpec((H * D,), lambda i: (0,))],
        out_specs=[pl.BlockSpec((B, D), lambda i: (i, 0)),
                   pl.BlockSpec((B, D), lambda i: (i, 0)),
                   pl.BlockSpec((8, D), lambda i: (0, 0))],
        out_shape=[jax.ShapeDtypeStruct((N, D), F32),
                   jax.ShapeDtypeStruct((N, D), F32),
                   jax.ShapeDtypeStruct((8, D), F32)])


def _tc_user(N, B=512):
    g = pl.cdiv(N, B)

    def body(xu_ref, sr_ref, cr_ref, su_ref, cu_ref,
             v0_ref, v1_ref, v2_ref, v3_ref,
             d0_ref, d1_ref, d2_ref, d3_ref,
             sg_ref, di_ref,
             wlr_ref, wrr_ref, wlu_ref, wru_ref, wsrc_ref, wsim_ref, b_ref,
             nu_ref, st_ref):
        i = pl.program_id(0)
        xu = xu_ref[...]
        mrec = sr_ref[...] / jnp.maximum(cr_ref[...], 1.0)[:, None]
        mus = su_ref[...] / jnp.maximum(cu_ref[...], 1.0)[:, None]
        nu = (jnp.dot(mrec, wlr_ref[...], preferred_element_type=F32)
              + jnp.dot(mus, wlu_ref[...], preferred_element_type=F32)
              + jnp.dot(xu, wrr_ref[...] + wru_ref[...],
                        preferred_element_type=F32)
              + b_ref[...][None, :])
        vs = (v0_ref, v1_ref, v2_ref, v3_ref)
        ds = (d0_ref, d1_ref, d2_ref, d3_ref)
        for h in range(H):
            vh = vs[h][...] / (ds[h][...] + 1e-16)[:, None]
            nu = nu + 0.25 * jnp.dot(vh, wsrc_ref[:, h * D:(h + 1) * D],
                                     preferred_element_type=F32)
        di = di_ref[...]
        gterm = di[:, None] * sg_ref[...] + (di * di)[:, None] * xu
        nu = nu + jnp.dot(gterm, wsim_ref[...], preferred_element_type=F32)
        nu_ref[...] = nu
        rvalid = (lax.broadcasted_iota(I32, (B, 1), 0) + i * B) < N
        num = jnp.where(rvalid, nu, 0.0)
        st = jnp.concatenate(
            [jnp.sum(num, axis=0, keepdims=True),
             jnp.sum(num * num, axis=0, keepdims=True),
             jnp.zeros((6, D), F32)], axis=0)
        @pl.when(i == 0)
        def _():
            st_ref[...] = jnp.zeros((8, D), F32)
        st_ref[...] = st_ref[...] + st

    full = lambda shape: pl.BlockSpec(shape, lambda i: tuple(0 for _ in shape))
    row2 = pl.BlockSpec((B, D), lambda i: (i, 0))
    row1 = pl.BlockSpec((B,), lambda i: (i,))
    return pl.pallas_call(
        body, grid=(g,),
        in_specs=[row2, row2, row1, row2, row1,
                  row2, row2, row2, row2,
                  row1, row1, row1, row1,
                  row2, row1,
                  full((D, D)), full((D, D)), full((D, D)), full((D, D)),
                  full((D, H * D)), full((D, D)), full((D,))],
        out_specs=[row2, full((8, D))],
        out_shape=[jax.ShapeDtypeStruct((N, D), F32),
                   jax.ShapeDtypeStruct((8, D), F32)])


def _tc_sage1(N, B=512):
    g = pl.cdiv(N, B)

    def body(x_ref, s_ref, c_ref, wl_ref, wr_ref, b_ref, o_ref, st_ref):
        i = pl.program_id(0)
        mean = s_ref[...] / jnp.maximum(c_ref[...], 1.0)[:, None]
        o = (jnp.dot(mean, wl_ref[...], preferred_element_type=F32)
             + jnp.dot(x_ref[...], wr_ref[...], preferred_element_type=F32)
             + b_ref[...][None, :])
        o_ref[...] = o
        rvalid = (lax.broadcasted_iota(I32, (B, 1), 0) + i * B) < N
        om = jnp.where(rvalid, o, 0.0)
        st = jnp.concatenate(
            [jnp.sum(om, axis=0, keepdims=True),
             jnp.sum(om * om, axis=0, keepdims=True),
             jnp.zeros((6, D), F32)], axis=0)
        @pl.when(i == 0)
        def _():
            st_ref[...] = jnp.zeros((8, D), F32)
        st_ref[...] = st_ref[...] + st

    full = lambda shape: pl.BlockSpec(shape, lambda i: tuple(0 for _ in shape))
    row2 = pl.BlockSpec((B, D), lambda i: (i, 0))
    row1 = pl.BlockSpec((B,), lambda i: (i,))
    return pl.pallas_call(
        body, grid=(g,),
        in_specs=[row2, row2, row1, full((D, D)), full((D, D)), full((D,))],
        out_specs=[row2, full((8, D))],
        out_shape=[jax.ShapeDtypeStruct((N, D), F32),
                   jax.ShapeDtypeStruct((8, D), F32)])


def _tc_bn(N, with_scale, B=1024):
    g = pl.cdiv(N, B)
    inv_n = 1.0 / N

    def body(*refs):
        if with_scale:
            x_ref, st_ref, g_ref, b_ref, di_ref, o_ref, og_ref = refs
        else:
            x_ref, st_ref, g_ref, b_ref, o_ref = refs
        mu = st_ref[0, :] * inv_n
        var = jnp.maximum(st_ref[1, :] * inv_n - mu * mu, 0.0)
        inv = lax.rsqrt(var + 1e-5)
        y = jnp.maximum(
            g_ref[...][None, :] * (x_ref[...] - mu[None, :]) * inv[None, :]
            + b_ref[...][None, :], 0.0)
        o_ref[...] = y
        if with_scale:
            og_ref[...] = di_ref[...][:, None] * y

    full = lambda shape: pl.BlockSpec(shape, lambda i: tuple(0 for _ in shape))
    row2 = pl.BlockSpec((B, D), lambda i: (i, 0))
    row1 = pl.BlockSpec((B,), lambda i: (i,))
    in_specs = [row2, full((8, D)), full((D,)), full((D,))]
    out_specs = [row2]
    out_shape = [jax.ShapeDtypeStruct((N, D), F32)]
    if with_scale:
        in_specs.append(row1)
        out_specs.append(row2)
        out_shape.append(jax.ShapeDtypeStruct((N, D), F32))
    return pl.pallas_call(body, grid=(g,), in_specs=in_specs,
                          out_specs=out_specs, out_shape=out_shape)


def _pad_edges(src, dst):
    e = src.shape[0]
    ep = _rup(e, 65536)
    src = jnp.pad(src, (0, ep - e))
    dst = jnp.pad(dst, (0, ep - e), constant_values=PADV)
    return src, dst, ep


def _seg(x, src, dst, ep, n_dst, rr, want_cnt):
    n_rng2 = _rup(pl.cdiv(n_dst, rr), NC)
    f = _sc_seg_sum(x.shape[0], ep, rr, n_rng2, want_cnt)
    res = f(x, src, dst)
    if want_cnt:
        out, cntp = res
        rd = rr + 16
        cnt = _tc_red(NS, n_rng2 * rd)(cntp.reshape(NS, n_rng2 * rd))
        cnt = cnt.reshape(n_rng2, rd)[:, :rr].reshape(-1)[:n_dst]
        return out[:n_dst], cnt
    return res[0][:n_dst]


def kernel(x_user, x_merchant, x_device, transacts_src, transacts_dst,
           receives_src, receives_dst, uses_src, uses_dst, used_by_src,
           used_by_dst, temporal_src, temporal_dst, similar_src, similar_dst,
           params):
    NU, NM, ND = x_user.shape[0], x_merchant.shape[0], x_device.shape[0]
    tr_s, tr_d, tr_ep = _pad_edges(transacts_src, transacts_dst)
    rc_s, rc_d, rc_ep = _pad_edges(receives_src, receives_dst)
    us_s, us_d, us_ep = _pad_edges(uses_src, uses_dst)
    ub_s, ub_d, ub_ep = _pad_edges(used_by_src, used_by_dst)
    tp_s, tp_d, tp_ep = _pad_edges(temporal_src, temporal_dst)
    sm_s, sm_d, sm_ep = _pad_edges(similar_src, similar_dst)

    R_U = 6144
    R_GAT = 1024

    n_rng2h = _rup(pl.cdiv(NU, R_U), NC)
    rdh = R_U + 16
    histp = _sc_hist(NU, sm_ep, R_U, n_rng2h)(sm_d)
    hist = _tc_red(NS, n_rng2h * rdh)(histp.reshape(NS, n_rng2h * rdh))
    hist = hist.reshape(n_rng2h, rdh)[:, :R_U].reshape(-1)[:NU]
    dinv = _tc_dinv(NU)(hist)

    xu, xm, xd = x_user, x_merchant, x_device
    xg = _tc_scale(NU)(dinv, xu)

    cnt_rc = cnt_ub = cnt_tr = cnt_us = None
    for li, layer in enumerate(params['layers']):
        if li == 0:
            s_rc, cnt_rc = _seg(xm, rc_s, rc_d, rc_ep, NU, R_U, True)
            s_ub, cnt_ub = _seg(xd, ub_s, ub_d, ub_ep, NU, R_U, True)
            s_tr, cnt_tr = _seg(xu, tr_s, tr_d, tr_ep, NM, 5120, True)
            s_us, cnt_us = _seg(xu, us_s, us_d, us_ep, ND, 5120, True)
        else:
            s_rc = _seg(xm, rc_s, rc_d, rc_ep, NU, R_U, False)
            s_ub = _seg(xd, ub_s, ub_d, ub_ep, NU, R_U, False)
            s_tr = _seg(xu, tr_s, tr_d, tr_ep, NM, 5120, False)
            s_us = _seg(xu, us_s, us_d, us_ep, ND, 5120, False)
        s_gcn = _seg(xg, sm_s, sm_d, sm_ep, NU, R_U, False)

        t = layer['temporal']
        as_tab, ad_tab, mrows = _tc_pre(NU)(
            xu, t['Wsrc'], t['Wdst'], t['att_src'].reshape(-1),
            t['att_dst'].reshape(-1))
        n_rng2 = _rup(pl.cdiv(NU, R_GAT), NC)
        v_out, denp = _sc_gat(NU, tp_ep, R_GAT, n_rng2)(
            xu, as_tab, ad_tab, mrows, tp_s, tp_d)
        rd = R_GAT + 16
        den = _tc_red(NS, n_rng2 * H * rd)(denp.reshape(NS, n_rng2 * H * rd))
        den = den.reshape(n_rng2, H, rd)
        dens = [den[:, h, :R_GAT].reshape(-1)[:NU] for h in range(H)]
        vhs = [v_out[h, :NU] for h in range(H)]

        r_p, u_p, g_p = layer['receives'], layer['used_by'], layer['similar']
        bias_u = r_p['b'] + u_p['b'] + t['b'] + g_p['b']
        nu, st_u = _tc_user(NU)(
            xu, s_rc, cnt_rc, s_ub, cnt_ub,
            vhs[0], vhs[1], vhs[2], vhs[3],
            dens[0], dens[1], dens[2], dens[3],
            s_gcn, dinv,
            r_p['Wl'], r_p['Wr'], u_p['Wl'], u_p['Wr'], t['Wsrc'], g_p['W'],
            bias_u)
        tr_p, us_p = layer['transacts'], layer['uses']
        nm, st_m = _tc_sage1(NM)(xm, s_tr, cnt_tr, tr_p['Wl'], tr_p['Wr'],
                                 tr_p['b'])
        nd, st_d = _tc_sage1(ND)(xd, s_us, cnt_us, us_p['Wl'], us_p['Wr'],
                                 us_p['b'])
        bn = layer['bn']
        xu, xg = _tc_bn(NU, True)(nu, st_u, bn['user']['g'], bn['user']['b'],
                                  dinv)
        xm = _tc_bn(NM, False)(nm, st_m, bn['merchant']['g'],
                               bn['merchant']['b'])[0]
        xd = _tc_bn(ND, False)(nd, st_d, bn['device']['g'],
                               bn['device']['b'])[0]
    return xu, xm, xd


# speedup vs baseline: 9.2526x; 9.2526x over previous
import functools

import jax
import jax.numpy as jnp
from jax import lax
from jax.experimental import pallas as pl
from jax.experimental.pallas import tpu as pltpu
from jax.experimental.pallas import tpu_sc as plsc

D = 128
H = 4
NC, NS, L = 2, 16, 16
BLK = 2048
F32 = jnp.float32
I32 = jnp.int32
PADV = 1 << 30

_MESH = plsc.VectorSubcoreMesh(core_axis_name="c", subcore_axis_name="s",
                               num_cores=NC, num_subcores=NS)
_SC_PARAMS = pltpu.CompilerParams(needs_layout_passes=False)


def _rup(x, m):
    return (x + m - 1) // m * m


@functools.cache
def _sc_seg_sum(N_src, E_pad, R, n_rng2, want_cnt):
    n_pass = n_rng2 // NC
    CH = E_pad // NS
    n_blk = CH // BLK
    K = 128
    CAP = BLK + K + 128
    RD = R + 16
    RT = R // NS
    outs = [jax.ShapeDtypeStruct((n_rng2 * R, D), F32)]
    if want_cnt:
        outs.append(jax.ShapeDtypeStruct((NS, n_rng2, RD), F32))
    scratch = [
        pltpu.VMEM((BLK,), I32),
        pltpu.VMEM((BLK,), I32),
        pltpu.VMEM((CAP,), I32),
        pltpu.VMEM((CAP,), I32),
        pltpu.VMEM((2, K), I32),
        pltpu.VMEM((2, K), I32),
        pltpu.VMEM((2, K, D), F32),
        pltpu.VMEM((8, D), F32),
        pltpu.VMEM((RD,), F32),
        pltpu.VMEM_SHARED((R + 1, D), F32),
        pltpu.SemaphoreType.DMA,
        pltpu.SemaphoreType.DMA,
        pltpu.SemaphoreType.DMA,
    ]

    def body(x_hbm, src_hbm, dst_hbm, *rest):
        if want_cnt:
            out_hbm, cnt_hbm = rest[0], rest[1]
            (sbuf, dbuf, gbuf, lbuf, gslot, didx, rows, zbuf, cntp, accum,
             sem_g0, sem_g1, sem_s) = rest[2:]
        else:
            out_hbm = rest[0]
            cnt_hbm = None
            (sbuf, dbuf, gbuf, lbuf, gslot, didx, rows, zbuf, cntp, accum,
             sem_g0, sem_g1, sem_s) = rest[1:]
        c = lax.axis_index("c")
        s = lax.axis_index("s")
        iot = lax.iota(I32, L)
        zv = jnp.zeros((L,), F32)
        ones = jnp.ones((L,), F32)

        def _zb(i, _):
            for k in range(D // L):
                zbuf[i, pl.ds(k * L, L)] = zv
            return 0
        lax.fori_loop(0, 8, _zb, 0)

        def drain_scatter():
            pltpu.make_async_copy(x_hbm.at[gslot.at[0]], rows.at[0],
                                  sem_s).wait()

        def drain_gather(slot):
            @pl.when(slot == 0)
            def _():
                pltpu.make_async_copy(x_hbm.at[gslot.at[0]], rows.at[0],
                                      sem_g0).wait()
            @pl.when(slot == 1)
            def _():
                pltpu.make_async_copy(x_hbm.at[gslot.at[1]], rows.at[1],
                                      sem_g1).wait()

        def fire_gather(slot):
            @pl.when(slot == 0)
            def _():
                pltpu.async_copy(x_hbm.at[gslot.at[0]], rows.at[0], sem_g0)
            @pl.when(slot == 1)
            def _():
                pltpu.async_copy(x_hbm.at[gslot.at[1]], rows.at[1], sem_g1)

        def fire_scatter(slot):
            @pl.when(slot == 0)
            def _():
                pltpu.async_copy(rows.at[0], accum.at[didx.at[0]], sem_s,
                                 add=True)
            @pl.when(slot == 1)
            def _():
                pltpu.async_copy(rows.at[1], accum.at[didx.at[1]], sem_s,
                                 add=True)

        def do_pass(p, _):
            r = p * NC + c
            lo = r * R

            def _za(i, _):
                pltpu.sync_copy(zbuf, accum.at[pl.ds(s * RT + i * 8, 8)])
                return 0
            lax.fori_loop(0, RT // 8, _za, 0)
            if want_cnt:
                def _zc(i, _):
                    cntp[pl.ds(i * L, L)] = zv
                    return 0
                lax.fori_loop(0, RD // L, _zc, 0)
            plsc.subcore_barrier()

            def do_blk(bi, st):
                off, pend, pslot, osc = st
                base = s * CH + bi * BLK
                pltpu.sync_copy(src_hbm.at[pl.ds(base, BLK)], sbuf)
                pltpu.sync_copy(dst_hbm.at[pl.ds(base, BLK)], dbuf)

                def filt(v, o):
                    dv = dbuf[pl.ds(v * L, L)]
                    sv = sbuf[pl.ds(v * L, L)]
                    m = (dv >= lo) & (dv < lo + R)
                    dl = jnp.where(m, dv - lo, R)
                    if want_cnt:
                        plsc.addupdate_scatter(cntp, [dl], ones, mask=m)
                    plsc.store_compressed(gbuf.at[pl.ds(o, L)], sv, mask=m)
                    plsc.store_compressed(lbuf.at[pl.ds(o, L)], dl, mask=m)
                    return o + jnp.sum(m.astype(I32))
                off = lax.fori_loop(0, BLK // L, filt, off)
                nfull = off // K

                def form(b, st2):
                    pend, pslot, osc = st2
                    slot = 1 - pslot
                    @pl.when(osc == 1)
                    def _():
                        drain_scatter()
                    for k in range(K // L):
                        gslot[slot, pl.ds(k * L, L)] = gbuf[
                            pl.ds(b * K + k * L, L)]
                        didx[slot, pl.ds(k * L, L)] = lbuf[
                            pl.ds(b * K + k * L, L)]
                    fire_gather(slot)
                    @pl.when(pend == 1)
                    def _():
                        drain_gather(pslot)
                        fire_scatter(pslot)
                    return (jnp.int32(1), slot, pend)
                pend, pslot, osc = lax.fori_loop(
                    0, nfull, form, (pend, pslot, osc))
                rem = off - nfull * K

                @pl.when(nfull > 0)
                def _():
                    def shift(v, _):
                        gbuf[pl.ds(v * L, L)] = gbuf[
                            pl.ds(nfull * K + v * L, L)]
                        lbuf[pl.ds(v * L, L)] = lbuf[
                            pl.ds(nfull * K + v * L, L)]
                        return 0
                    lax.fori_loop(0, (rem + L - 1) // L, shift, 0)
                return (rem, pend, pslot, osc)
            off, pend, pslot, osc = lax.fori_loop(
                0, n_blk, do_blk,
                (jnp.int32(0), jnp.int32(0), jnp.int32(0), jnp.int32(0)))

            @pl.when(pend == 1)
            def _():
                drain_gather(pslot)
                fire_scatter(pslot)

            @pl.when((osc + pend) >= 1)
            def _():
                drain_scatter()

            @pl.when((osc + pend) >= 2)
            def _():
                drain_scatter()

            @pl.when(off > 0)
            def _():
                def padf(v, _):
                    sel = (iot + v * L) >= off
                    gv = gbuf[pl.ds(v * L, L)]
                    lv = lbuf[pl.ds(v * L, L)]
                    gbuf[pl.ds(v * L, L)] = jnp.where(sel, 0, gv)
                    lbuf[pl.ds(v * L, L)] = jnp.where(sel, R, lv)
                    return 0
                lax.fori_loop(off // L, K // L, padf, 0)
                for k in range(K // L):
                    gslot[0, pl.ds(k * L, L)] = gbuf[pl.ds(k * L, L)]
                    didx[0, pl.ds(k * L, L)] = lbuf[pl.ds(k * L, L)]
                pltpu.async_copy(x_hbm.at[gslot.at[0]], rows.at[0],
                                 sem_g0).wait()
                pltpu.sync_copy(rows.at[0], accum.at[didx.at[0]], add=True)
            plsc.subcore_barrier()
            pltpu.sync_copy(accum.at[pl.ds(s * RT, RT)],
                            out_hbm.at[pl.ds(r * R + s * RT, RT)])
            if want_cnt:
                pltpu.sync_copy(cntp, cnt_hbm.at[s, r])
            return 0
        lax.fori_loop(0, n_pass, do_pass, 0)

    return pl.kernel(body, out_type=tuple(outs), mesh=_MESH,
                     scratch_types=scratch, compiler_params=_SC_PARAMS)


@functools.cache
def _sc_hist(N_dst, E_pad, R, n_rng2):
    n_pass = n_rng2 // NC
    CH = E_pad // NS
    n_blk = CH // BLK
    RD = R + 16
    scratch = [
        pltpu.VMEM((BLK,), I32),
        pltpu.VMEM((RD,), F32),
    ]

    def body(dst_hbm, out_hbm, dbuf, cntp):
        c = lax.axis_index("c")
        s = lax.axis_index("s")
        zv = jnp.zeros((L,), F32)
        ones = jnp.ones((L,), F32)

        def do_pass(p, _):
            r = p * NC + c
            lo = r * R

            def _zc(i, _):
                cntp[pl.ds(i * L, L)] = zv
                return 0
            lax.fori_loop(0, RD // L, _zc, 0)

            def do_blk(bi, _):
                base = s * CH + bi * BLK
                pltpu.sync_copy(dst_hbm.at[pl.ds(base, BLK)], dbuf)

                def inner(v, _):
                    dv = dbuf[pl.ds(v * L, L)]
                    m = (dv >= lo) & (dv < lo + R)
                    dl = jnp.where(m, dv - lo, R)
                    plsc.addupdate_scatter(cntp, [dl], ones, mask=m)
                    return 0
                lax.fori_loop(0, BLK // L, inner, 0)
                return 0
            lax.fori_loop(0, n_blk, do_blk, 0)
            pltpu.sync_copy(cntp, out_hbm.at[s, r])
            return 0
        lax.fori_loop(0, n_pass, do_pass, 0)

    return pl.kernel(
        body, out_type=jax.ShapeDtypeStruct((NS, n_rng2, RD), F32),
        mesh=_MESH, scratch_types=scratch, compiler_params=_SC_PARAMS)


@functools.cache
def _sc_gat(N, E_pad, R, n_rng2):
    BLKG = 1024
    n_pass = n_rng2 // NC
    CH = E_pad // NS
    n_blk = CH // BLKG
    K = 64
    CAP = BLKG + K + 128
    RD = R + 16
    RT = R // NS
    RP = R + 1
    outs = [jax.ShapeDtypeStruct((H, n_rng2 * R, D), F32),
            jax.ShapeDtypeStruct((NS, n_rng2, H * RD), F32)]
    scratch = [
        pltpu.VMEM((BLKG,), I32),
        pltpu.VMEM((BLKG,), I32),
        pltpu.VMEM((CAP,), I32),
        pltpu.VMEM((CAP,), I32),
        pltpu.VMEM((K,), I32),
        pltpu.VMEM((1, K), I32),
        pltpu.VMEM((K, D), F32),
        pltpu.VMEM((K, D), F32),
        pltpu.VMEM((K, D), F32),
        pltpu.VMEM((K, D), F32),
        pltpu.VMEM((8, D), F32),
        pltpu.VMEM((H * RD,), F32),
        pltpu.VMEM((2, D), F32),
        pltpu.VMEM_SHARED((H * RP, D), F32),
        pltpu.SemaphoreType.DMA,
    ]

    def body(x_hbm, as_hbm, ad_hbm, m_hbm, src_hbm, dst_hbm, out_hbm, cnt_hbm,
             sbuf, dbuf, gbuf, lbuf, aidx, didx, rows, wbuf, asb, adb, zbuf,
             cntp, mb, accum, sem):
        c = lax.axis_index("c")
        s = lax.axis_index("s")
        iot = lax.iota(I32, L)
        zv = jnp.zeros((L,), F32)

        def _zb(i, _):
            for k in range(D // L):
                zbuf[i, pl.ds(k * L, L)] = zv
            return 0
        lax.fori_loop(0, 8, _zb, 0)
        pltpu.sync_copy(m_hbm.at[pl.ds(0, 2)], mb)

        def do_pass(p, _):
            r = p * NC + c
            lo = r * R
            for h in range(H):
                def _za(i, _, h=h):
                    pltpu.sync_copy(
                        zbuf, accum.at[pl.ds(h * RP + s * RT + i * 8, 8)])
                    return 0
                lax.fori_loop(0, RT // 8, _za, 0)

            def _zc(i, _):
                cntp[pl.ds(i * L, L)] = zv
                return 0
            lax.fori_loop(0, (H * RD) // L, _zc, 0)
            plsc.subcore_barrier()

            def proc(bo):
                for k in range(K // L):
                    aidx[pl.ds(k * L, L)] = jnp.minimum(
                        lbuf[pl.ds(bo + k * L, L)] + lo, N - 1)
                pltpu.async_copy(
                    x_hbm.at[gbuf.at[pl.ds(bo, K)]], rows, sem).wait()
                pltpu.async_copy(
                    as_hbm.at[gbuf.at[pl.ds(bo, K)]], asb, sem).wait()
                pltpu.async_copy(ad_hbm.at[aidx], adb, sem).wait()
                for g in range(K // L):
                    rowi = iot + g * L
                    dlv = lbuf[pl.ds(bo + g * L, L)]
                    for h in range(H):
                        hc = jnp.full((L,), h * L, I32)
                        av = plsc.load_gather(asb, [rowi, hc])
                        bv = plsc.load_gather(adb, [rowi, hc])
                        vv = av + bv
                        e = jnp.maximum(vv, 0.2 * vv)
                        w = jnp.exp(e - (mb[0, pl.ds(h * L, L)]
                                         + mb[1, pl.ds(h * L, L)]))
                        plsc.addupdate_scatter(cntp, [dlv + h * RD], w)
                for h in range(H):
                    def per_edge(j, _, h=h):
                        va = (asb[j, pl.ds(h * L, L)]
                              + adb[j, pl.ds(h * L, L)])
                        e = jnp.maximum(va, 0.2 * va)
                        w = jnp.exp(e - (mb[0, pl.ds(h * L, L)]
                                         + mb[1, pl.ds(h * L, L)]))
                        for k in range(D // L):
                            wbuf[j, pl.ds(k * L, L)] = (
                                rows[j, pl.ds(k * L, L)] * w)
                        return 0
                    lax.fori_loop(0, K, per_edge, 0)
                    for k in range(K // L):
                        didx[0, pl.ds(k * L, L)] = (
                            lbuf[pl.ds(bo + k * L, L)] + h * RP)
                    pltpu.sync_copy(wbuf, accum.at[didx.at[0]], add=True)

            def do_blk(bi, off):
                base = s * CH + bi * BLKG
                pltpu.sync_copy(src_hbm.at[pl.ds(base, BLKG)], sbuf)
                pltpu.sync_copy(dst_hbm.at[pl.ds(base, BLKG)], dbuf)

                def filt(v, o):
                    dv = dbuf[pl.ds(v * L, L)]
                    sv = sbuf[pl.ds(v * L, L)]
                    m = (dv >= lo) & (dv < lo + R)
                    dl = jnp.where(m, dv - lo, R)
                    plsc.store_compressed(gbuf.at[pl.ds(o, L)], sv, mask=m)
                    plsc.store_compressed(lbuf.at[pl.ds(o, L)], dl, mask=m)
                    return o + jnp.sum(m.astype(I32))
                off = lax.fori_loop(0, BLKG // L, filt, off)
                nfull = off // K

                def form(b, _):
                    proc(b * K)
                    return 0
                lax.fori_loop(0, nfull, form, 0)
                rem = off - nfull * K

                @pl.when(nfull > 0)
                def _():
                    def shift(v, _):
                        gbuf[pl.ds(v * L, L)] = gbuf[
                            pl.ds(nfull * K + v * L, L)]
                        lbuf[pl.ds(v * L, L)] = lbuf[
                            pl.ds(nfull * K + v * L, L)]
                        return 0
                    lax.fori_loop(0, (rem + L - 1) // L, shift, 0)
                return rem
            off = lax.fori_loop(0, n_blk, do_blk, jnp.int32(0))

            @pl.when(off > 0)
            def _():
                def padf(v, _):
                    sel = (iot + v * L) >= off
                    gv = gbuf[pl.ds(v * L, L)]
                    lv = lbuf[pl.ds(v * L, L)]
                    gbuf[pl.ds(v * L, L)] = jnp.where(sel, 0, gv)
                    lbuf[pl.ds(v * L, L)] = jnp.where(sel, R, lv)
                    return 0
                lax.fori_loop(off // L, K // L, padf, 0)
                proc(0)
            plsc.subcore_barrier()
            for h in range(H):
                pltpu.sync_copy(accum.at[pl.ds(h * RP + s * RT, RT)],
                                out_hbm.at[h, pl.ds(r * R + s * RT, RT)])
            pltpu.sync_copy(cntp, cnt_hbm.at[s, r])
            return 0
        lax.fori_loop(0, n_pass, do_pass, 0)

    return pl.kernel(body, out_type=tuple(outs), mesh=_MESH,
                     scratch_types=scratch, compiler_params=_SC_PARAMS)


def _tc_red(T, X, BX=8192):
    def body(p_ref, o_ref):
        o_ref[...] = jnp.sum(p_ref[...], axis=0)
    return pl.pallas_call(
        body, grid=(pl.cdiv(X, BX),),
        in_specs=[pl.BlockSpec((T, BX), lambda i: (0, i))],
        out_specs=pl.BlockSpec((BX,), lambda i: (i,)),
        out_shape=jax.ShapeDtypeStruct((X,), F32))


def _tc_dinv(X, BX=8192):
    def body(h_ref, o_ref):
        o_ref[...] = lax.rsqrt(h_ref[...] + 1.0)
    return pl.pallas_call(
        body, grid=(pl.cdiv(X, BX),),
        in_specs=[pl.BlockSpec((BX,), lambda i: (i,))],
        out_specs=pl.BlockSpec((BX,), lambda i: (i,)),
        out_shape=jax.ShapeDtypeStruct((X,), F32))


def _tc_scale(N, B=1024):
    def body(d_ref, x_ref, o_ref):
        o_ref[...] = d_ref[...][:, None] * x_ref[...]
    return pl.pallas_call(
        body, grid=(pl.cdiv(N, B),),
        in_specs=[pl.BlockSpec((B,), lambda i: (i,)),
                  pl.BlockSpec((B, D), lambda i: (i, 0))],
        out_specs=pl.BlockSpec((B, D), lambda i: (i, 0)),
        out_shape=jax.ShapeDtypeStruct((N, D), F32))


def _tc_pre(N, B=512):
    g = pl.cdiv(N, B)

    def body(x_ref, ws_ref, wd_ref, ats_ref, atd_ref, as_ref, ad_ref, m_ref):
        i = pl.program_id(0)
        x = x_ref[...]
        rvalid = (lax.broadcasted_iota(I32, (B, 1), 0) + i * B) < N

        def tab(wref, atref):
            cols = []
            for h in range(H):
                wh = wref[:, h * D:(h + 1) * D]
                cols.append(jnp.sum(wh * atref[h * D:(h + 1) * D][None, :],
                                    axis=1, keepdims=True))
            A = jnp.concatenate(cols, axis=1)
            a = jnp.dot(x, A, preferred_element_type=F32)
            parts = [jnp.broadcast_to(a[:, h:h + 1], (B, L)) for h in range(H)]
            parts.append(jnp.zeros((B, D - H * L), F32))
            am = jnp.where(rvalid, a, -1e30)
            mx = jnp.max(am, axis=0)
            mparts = [jnp.broadcast_to(mx[h:h + 1], (L,)) for h in range(H)]
            mparts.append(jnp.zeros((D - H * L,), F32))
            return jnp.concatenate(parts, axis=1), jnp.concatenate(mparts)

        as_tab, ms = tab(ws_ref, ats_ref)
        ad_tab, md = tab(wd_ref, atd_ref)
        as_ref[...] = as_tab
        ad_ref[...] = ad_tab
        mnew = jnp.concatenate(
            [ms[None, :], md[None, :], jnp.full((6, D), -1e30, F32)], axis=0)
        @pl.when(i == 0)
        def _():
            m_ref[...] = jnp.full((8, D), -1e30, F32)
        m_ref[...] = jnp.maximum(m_ref[...], mnew)

    return pl.pallas_call(
        body, grid=(g,),
        in_specs=[pl.BlockSpec((B, D), lambda i: (i, 0)),
                  pl.BlockSpec((D, H * D), lambda i: (0, 0)),
                  pl.BlockSpec((D, H * D), lambda i: (0, 0)),
                  pl.BlockSpec((H * D,), lambda i: (0,)),
                  pl.BlockSpec((H * D,), lambda i: (0,))],
        out_specs=[pl.BlockSpec((B, D), lambda i: (i, 0)),
                   pl.BlockSpec((B, D), lambda i: (i, 0)),
                   pl.BlockSpec((8, D), lambda i: (0, 0))],
        out_shape=[jax.ShapeDtypeStruct((N, D), F32),
                   jax.ShapeDtypeStruct((N, D), F32),
                   jax.ShapeDtypeStruct((8, D), F32)])


def _tc_user(N, B=512):
    g = pl.cdiv(N, B)

    def body(xu_ref, sr_ref, cr_ref, su_ref, cu_ref,
             v0_ref, v1_ref, v2_ref, v3_ref,
             d0_ref, d1_ref, d2_ref, d3_ref,
             sg_ref, di_ref,
             wlr_ref, wrr_ref, wlu_ref, wru_ref, wsrc_ref, wsim_ref, b_ref,
             nu_ref, st_ref):
        i = pl.program_id(0)
        xu = xu_ref[...]
        mrec = sr_ref[...] / jnp.maximum(cr_ref[...], 1.0)[:, None]
        mus = su_ref[...] / jnp.maximum(cu_ref[...], 1.0)[:, None]
        nu = (jnp.dot(mrec, wlr_ref[...], preferred_element_type=F32)
              + jnp.dot(mus, wlu_ref[...], preferred_element_type=F32)
              + jnp.dot(xu, wrr_ref[...] + wru_ref[...],
                        preferred_element_type=F32)
              + b_ref[...][None, :])
        vs = (v0_ref, v1_ref, v2_ref, v3_ref)
        ds = (d0_ref, d1_ref, d2_ref, d3_ref)
        for h in range(H):
            vh = vs[h][...] / (ds[h][...] + 1e-16)[:, None]
            nu = nu + 0.25 * jnp.dot(vh, wsrc_ref[:, h * D:(h + 1) * D],
                                     preferred_element_type=F32)
        di = di_ref[...]
        gterm = di[:, None] * sg_ref[...] + (di * di)[:, None] * xu
        nu = nu + jnp.dot(gterm, wsim_ref[...], preferred_element_type=F32)
        nu_ref[...] = nu
        rvalid = (lax.broadcasted_iota(I32, (B, 1), 0) + i * B) < N
        num = jnp.where(rvalid, nu, 0.0)
        st = jnp.concatenate(
            [jnp.sum(num, axis=0, keepdims=True),
             jnp.sum(num * num, axis=0, keepdims=True),
             jnp.zeros((6, D), F32)], axis=0)
        @pl.when(i == 0)
        def _():
            st_ref[...] = jnp.zeros((8, D), F32)
        st_ref[...] = st_ref[...] + st

    full = lambda shape: pl.BlockSpec(shape, lambda i: tuple(0 for _ in shape))
    row2 = pl.BlockSpec((B, D), lambda i: (i, 0))
    row1 = pl.BlockSpec((B,), lambda i: (i,))
    return pl.pallas_call(
        body, grid=(g,),
        in_specs=[row2, row2, row1, row2, row1,
                  row2, row2, row2, row2,
                  row1, row1, row1, row1,
                  row2, row1,
                  full((D, D)), full((D, D)), full((D, D)), full((D, D)),
                  full((D, H * D)), full((D, D)), full((D,))],
        out_specs=[row2, full((8, D))],
        out_shape=[jax.ShapeDtypeStruct((N, D), F32),
                   jax.ShapeDtypeStruct((8, D), F32)])


def _tc_sage1(N, B=512):
    g = pl.cdiv(N, B)

    def body(x_ref, s_ref, c_ref, wl_ref, wr_ref, b_ref, o_ref, st_ref):
        i = pl.program_id(0)
        mean = s_ref[...] / jnp.maximum(c_ref[...], 1.0)[:, None]
        o = (jnp.dot(mean, wl_ref[...], preferred_element_type=F32)
             + jnp.dot(x_ref[...], wr_ref[...], preferred_element_type=F32)
             + b_ref[...][None, :])
        o_ref[...] = o
        rvalid = (lax.broadcasted_iota(I32, (B, 1), 0) + i * B) < N
        om = jnp.where(rvalid, o, 0.0)
        st = jnp.concatenate(
            [jnp.sum(om, axis=0, keepdims=True),
             jnp.sum(om * om, axis=0, keepdims=True),
             jnp.zeros((6, D), F32)], axis=0)
        @pl.when(i == 0)
        def _():
            st_ref[...] = jnp.zeros((8, D), F32)
        st_ref[...] = st_ref[...] + st

    full = lambda shape: pl.BlockSpec(shape, lambda i: tuple(0 for _ in shape))
    row2 = pl.BlockSpec((B, D), lambda i: (i, 0))
    row1 = pl.BlockSpec((B,), lambda i: (i,))
    return pl.pallas_call(
        body, grid=(g,),
        in_specs=[row2, row2, row1, full((D, D)), full((D, D)), full((D,))],
        out_specs=[row2, full((8, D))],
        out_shape=[jax.ShapeDtypeStruct((N, D), F32),
                   jax.ShapeDtypeStruct((8, D), F32)])


def _tc_bn(N, with_scale, B=1024):
    g = pl.cdiv(N, B)
    inv_n = 1.0 / N

    def body(*refs):
        if with_scale:
            x_ref, st_ref, g_ref, b_ref, di_ref, o_ref, og_ref = refs
        else:
            x_ref, st_ref, g_ref, b_ref, o_ref = refs
        mu = st_ref[0, :] * inv_n
        var = jnp.maximum(st_ref[1, :] * inv_n - mu * mu, 0.0)
        inv = lax.rsqrt(var + 1e-5)
        y = jnp.maximum(
            g_ref[...][None, :] * (x_ref[...] - mu[None, :]) * inv[None, :]
            + b_ref[...][None, :], 0.0)
        o_ref[...] = y
        if with_scale:
            og_ref[...] = di_ref[...][:, None] * y

    full = lambda shape: pl.BlockSpec(shape, lambda i: tuple(0 for _ in shape))
    row2 = pl.BlockSpec((B, D), lambda i: (i, 0))
    row1 = pl.BlockSpec((B,), lambda i: (i,))
    in_specs = [row2, full((8, D)), full((D,)), full((D,))]
    out_specs = [row2]
    out_shape = [jax.ShapeDtypeStruct((N, D), F32)]
    if with_scale:
        in_specs.append(row1)
        out_specs.append(row2)
        out_shape.append(jax.ShapeDtypeStruct((N, D), F32))
    return pl.pallas_call(body, grid=(g,), in_specs=in_specs,
                          out_specs=out_specs, out_shape=out_shape)


def _pad_edges(src, dst):
    e = src.shape[0]
    ep = _rup(e, 65536)
    src = jnp.pad(src, (0, ep - e))
    dst = jnp.pad(dst, (0, ep - e), constant_values=PADV)
    return src, dst, ep


def _seg(x, src, dst, ep, n_dst, rr, want_cnt):
    n_rng2 = _rup(pl.cdiv(n_dst, rr), NC)
    f = _sc_seg_sum(x.shape[0], ep, rr, n_rng2, want_cnt)
    res = f(x, src, dst)
    if want_cnt:
        out, cntp = res
        rd = rr + 16
        cnt = _tc_red(NS, n_rng2 * rd)(cntp.reshape(NS, n_rng2 * rd))
        cnt = cnt.reshape(n_rng2, rd)[:, :rr].reshape(-1)[:n_dst]
        return out[:n_dst], cnt
    return res[0][:n_dst]


def kernel(x_user, x_merchant, x_device, transacts_src, transacts_dst,
           receives_src, receives_dst, uses_src, uses_dst, used_by_src,
           used_by_dst, temporal_src, temporal_dst, similar_src, similar_dst,
           params):
    NU, NM, ND = x_user.shape[0], x_merchant.shape[0], x_device.shape[0]
    tr_s, tr_d, tr_ep = _pad_edges(transacts_src, transacts_dst)
    rc_s, rc_d, rc_ep = _pad_edges(receives_src, receives_dst)
    us_s, us_d, us_ep = _pad_edges(uses_src, uses_dst)
    ub_s, ub_d, ub_ep = _pad_edges(used_by_src, used_by_dst)
    tp_s, tp_d, tp_ep = _pad_edges(temporal_src, temporal_dst)
    sm_s, sm_d, sm_ep = _pad_edges(similar_src, similar_dst)

    R_U = 5120
    R_GAT = 1024

    n_rng2h = _rup(pl.cdiv(NU, R_U), NC)
    rdh = R_U + 16
    histp = _sc_hist(NU, sm_ep, R_U, n_rng2h)(sm_d)
    hist = _tc_red(NS, n_rng2h * rdh)(histp.reshape(NS, n_rng2h * rdh))
    hist = hist.reshape(n_rng2h, rdh)[:, :R_U].reshape(-1)[:NU]
    dinv = _tc_dinv(NU)(hist)

    xu, xm, xd = x_user, x_merchant, x_device
    xg = _tc_scale(NU)(dinv, xu)

    cnt_rc = cnt_ub = cnt_tr = cnt_us = None
    for li, layer in enumerate(params['layers']):
        if li == 0:
            s_rc, cnt_rc = _seg(xm, rc_s, rc_d, rc_ep, NU, R_U, True)
            s_ub, cnt_ub = _seg(xd, ub_s, ub_d, ub_ep, NU, R_U, True)
            s_tr, cnt_tr = _seg(xu, tr_s, tr_d, tr_ep, NM, 5120, True)
            s_us, cnt_us = _seg(xu, us_s, us_d, us_ep, ND, 5120, True)
        else:
            s_rc = _seg(xm, rc_s, rc_d, rc_ep, NU, R_U, False)
            s_ub = _seg(xd, ub_s, ub_d, ub_ep, NU, R_U, False)
            s_tr = _seg(xu, tr_s, tr_d, tr_ep, NM, 5120, False)
            s_us = _seg(xu, us_s, us_d, us_ep, ND, 5120, False)
        s_gcn = _seg(xg, sm_s, sm_d, sm_ep, NU, R_U, False)

        t = layer['temporal']
        as_tab, ad_tab, mrows = _tc_pre(NU)(
            xu, t['Wsrc'], t['Wdst'], t['att_src'].reshape(-1),
            t['att_dst'].reshape(-1))
        n_rng2 = _rup(pl.cdiv(NU, R_GAT), NC)
        v_out, denp = _sc_gat(NU, tp_ep, R_GAT, n_rng2)(
            xu, as_tab, ad_tab, mrows, tp_s, tp_d)
        rd = R_GAT + 16
        den = _tc_red(NS, n_rng2 * H * rd)(denp.reshape(NS, n_rng2 * H * rd))
        den = den.reshape(n_rng2, H, rd)
        dens = [den[:, h, :R_GAT].reshape(-1)[:NU] for h in range(H)]
        vhs = [v_out[h, :NU] for h in range(H)]

        r_p, u_p, g_p = layer['receives'], layer['used_by'], layer['similar']
        bias_u = r_p['b'] + u_p['b'] + t['b'] + g_p['b']
        nu, st_u = _tc_user(NU)(
            xu, s_rc, cnt_rc, s_ub, cnt_ub,
            vhs[0], vhs[1], vhs[2], vhs[3],
            dens[0], dens[1], dens[2], dens[3],
            s_gcn, dinv,
            r_p['Wl'], r_p['Wr'], u_p['Wl'], u_p['Wr'], t['Wsrc'], g_p['W'],
            bias_u)
        tr_p, us_p = layer['transacts'], layer['uses']
        nm, st_m = _tc_sage1(NM)(xm, s_tr, cnt_tr, tr_p['Wl'], tr_p['Wr'],
                                 tr_p['b'])
        nd, st_d = _tc_sage1(ND)(xd, s_us, cnt_us, us_p['Wl'], us_p['Wr'],
                                 us_p['b'])
        bn = layer['bn']
        xu, xg = _tc_bn(NU, True)(nu, st_u, bn['user']['g'], bn['user']['b'],
                                  dinv)
        xm = _tc_bn(NM, False)(nm, st_m, bn['merchant']['g'],
                               bn['merchant']['b'])[0]
        xd = _tc_bn(ND, False)(nd, st_d, bn['device']['g'],
                               bn['device']['b'])[0]
    return xu, xm, xd
